# Initial kernel scaffold; baseline (speedup 1.0000x reference)
import jax
import jax.numpy as jnp
from jax.experimental import pallas as pl


def kernel(x_in, edge_index, edge_attr, params):
    raise NotImplementedError("write your pallas kernel here")


# speedup vs baseline: 3.6115x; 3.6115x over previous
import functools

import jax
import jax.numpy as jnp
from jax import lax
from jax.experimental import pallas as pl
from jax.experimental.pallas import tpu as pltpu
from jax.experimental.pallas import tpu_sc as plsc

N = 10000
E = 320000
D = 128
ED = 16
OD = 128
H = OD // 2

NC = 2
NS = 16
NW = NC * NS
EW = E // NW
C = 125
NCH = EW // C
RT = N // NS
RZ = RT // 5

_BN = 2000
_BE = 8000


def _full(shape):
  return pl.BlockSpec(shape, lambda i: tuple(0 for _ in shape))


def _ln(x, g, b):
  mu = jnp.mean(x, axis=-1, keepdims=True)
  var = jnp.mean((x - mu) ** 2, axis=-1, keepdims=True)
  return (x - mu) * lax.rsqrt(var + 1e-6) * g + b


def _node_pre_body(x_ref, dummy_ref, g_ref, b_ref, wd_ref, b1_ref, ws_ref,
                   skw_ref, skb_ref, gw_ref, gb_ref,
                   pa_ref, pb_ref, skip_ref, gate_ref):
  x = x_ref[...]
  inval = x[:, 0:1] == -999.0
  x = jnp.where(inval, dummy_ref[...], x)
  xn = _ln(x, g_ref[...], b_ref[...])
  pa_ref[...] = jnp.dot(xn, wd_ref[...], preferred_element_type=jnp.float32) + b1_ref[...]
  pb_ref[...] = jnp.dot(xn, ws_ref[...], preferred_element_type=jnp.float32)
  sk = jnp.dot(xn, skw_ref[...], preferred_element_type=jnp.float32) + skb_ref[...]
  skip_ref[...] = sk
  gate_ref[...] = jax.nn.sigmoid(
      jnp.dot(sk, gw_ref[...], preferred_element_type=jnp.float32) + gb_ref[...])


def _edge1_body(ea_ref, pn1_ref, elng_ref, elnb_ref, w1_ref, b1_ref, w2_ref,
                b2_ref, w3_ref, b3_ref, cw1_ref, cb1_ref, cw2_ref, cb2_ref,
                we_ref, c1w2_ref, c1b2_ref, c1w3_ref, c1b3_ref,
                eenc_ref, h1_ref):
  ea = ea_ref[...]
  h = _ln(ea, elng_ref[...], elnb_ref[...])
  h = jax.nn.relu(jnp.dot(h, w1_ref[...], preferred_element_type=jnp.float32) + b1_ref[...])
  h = jax.nn.relu(jnp.dot(h, w2_ref[...], preferred_element_type=jnp.float32) + b2_ref[...])
  eenc = jnp.dot(h, w3_ref[...], preferred_element_type=jnp.float32) + b3_ref[...]
  r = jax.nn.relu(jnp.dot(ea, cw1_ref[...], preferred_element_type=jnp.float32) + cb1_ref[...])
  w = jax.nn.sigmoid(
      jnp.sum(r * cw2_ref[...], axis=-1, keepdims=True) + cb2_ref[...])
  eenc = eenc * w
  eenc_ref[...] = eenc
  m = jax.nn.relu(
      pn1_ref[...] + jnp.dot(eenc, we_ref[...], preferred_element_type=jnp.float32))
  m = jax.nn.relu(jnp.dot(m, c1w2_ref[...], preferred_element_type=jnp.float32) + c1b2_ref[...])
  h1_ref[...] = jnp.dot(m, c1w3_ref[...], preferred_element_type=jnp.float32) + c1b3_ref[...]


def _node_mid_body(s1_ref, cnt_ref, g_ref, b_ref, wd_ref, b1_ref, ws_ref,
                   pa_ref, pb_ref):
  cnt = jnp.maximum(cnt_ref[0][:, 0:1] + cnt_ref[1][:, 0:1], 1.0)
  x1 = (s1_ref[0] + s1_ref[1]) / cnt
  x1 = _ln(x1, g_ref[...], b_ref[...])
  x1 = jnp.where(x1 > 0, x1, 0.01 * x1)
  pa_ref[...] = jnp.dot(x1, wd_ref[...], preferred_element_type=jnp.float32) + b1_ref[...]
  pb_ref[...] = jnp.dot(x1, ws_ref[...], preferred_element_type=jnp.float32)


def _edge2_body(pn2_ref, eenc_ref, we_ref, w2_ref, b2_ref, w3_ref, b3_ref,
                h2_ref):
  m = jax.nn.relu(
      pn2_ref[...]
      + jnp.dot(eenc_ref[...], we_ref[...], preferred_element_type=jnp.float32))
  m = jax.nn.relu(jnp.dot(m, w2_ref[...], preferred_element_type=jnp.float32) + b2_ref[...])
  h2_ref[...] = jnp.dot(m, w3_ref[...], preferred_element_type=jnp.float32) + b3_ref[...]


def _elu(x):
  return jnp.where(x > 0, x, jnp.exp(jnp.minimum(x, 0.0)) - 1.0)


def _node_final_body(s2_ref, cnt_ref, ef_ref, skip_ref, gate_ref, g_ref,
                     b_ref, w1_ref, b1_ref, w2_ref, b2_ref, w3_ref, b3_ref,
                     xfc_ref, np_ref):
  cnt = jnp.maximum(cnt_ref[0][:, 0:1] + cnt_ref[1][:, 0:1], 1.0)
  x2 = (s2_ref[0] + s2_ref[1]) / cnt
  x2 = jax.nn.relu(_ln(x2, g_ref[...], b_ref[...]))
  gate = gate_ref[...]
  xf = gate * skip_ref[...] + (1.0 - gate) * x2
  efm = (ef_ref[0] + ef_ref[1]) / cnt
  xfc = jnp.concatenate([xf, efm], axis=1)
  xfc_ref[...] = xfc
  h = _elu(jnp.dot(xfc, w1_ref[...], preferred_element_type=jnp.float32) + b1_ref[...])
  h = _elu(jnp.dot(h, w2_ref[...], preferred_element_type=jnp.float32) + b2_ref[...])
  np_ref[...] = jnp.sum(h * w3_ref[...], axis=-1, keepdims=True) + b3_ref[...]


def _mesh():
  return plsc.VectorSubcoreMesh(core_axis_name="c", subcore_axis_name="s")


def _sc_gather(K):
  RPS = 250 if K == 64 else 125
  CPS = RPS // C
  NSETS = EW // RPS

  @functools.partial(
      pl.kernel,
      mesh=_mesh(),
      compiler_params=pltpu.CompilerParams(use_tc_tiling_on_sc=False),
      out_type=jax.ShapeDtypeStruct((E, K), jnp.float32),
      scratch_types=[
          pltpu.VMEM((NCH, C), jnp.int32),
          pltpu.VMEM((NCH, C), jnp.int32),
          pltpu.VMEM((RPS, K), jnp.float32),
          pltpu.VMEM((RPS, K), jnp.float32),
          pltpu.VMEM((RPS, K), jnp.float32),
          pltpu.VMEM((RPS, K), jnp.float32),
          pltpu.SemaphoreType.DMA,
          pltpu.SemaphoreType.DMA,
          pltpu.SemaphoreType.DMA,
          pltpu.SemaphoreType.DMA,
      ],
  )
  def k(pa_hbm, pb_hbm, dsti_hbm, srci_hbm, out_hbm,
        idxd, idxs, ra0, rb0, ra1, rb1, sa0, sb0, sa1, sb1):
    w = lax.axis_index("s") * NC + lax.axis_index("c")
    base_w = w * EW
    pltpu.sync_copy(dsti_hbm.at[w], idxd)
    pltpu.sync_copy(srci_hbm.at[w], idxs)

    def start(i, ra, rb, sa, sb):
      cps = []
      for t in range(CPS):
        rows = pl.ds(t * C, C)
        cps.append(pltpu.async_copy(pa_hbm.at[idxd.at[i * CPS + t]],
                                    ra.at[rows], sa))
        cps.append(pltpu.async_copy(pb_hbm.at[idxs.at[i * CPS + t]],
                                    rb.at[rows], sb))
      return cps

    def process(i, ra, rb):
      @pl.loop(0, RPS)
      def _(r):
        for l in range(K // 16):
          sl = pl.ds(l * 16, 16)
          ra[r, sl] = ra[r, sl] + rb[r, sl]
      pltpu.sync_copy(ra, out_hbm.at[pl.ds(base_w + i * RPS, RPS)])

    @pl.loop(0, NSETS, step=2)
    def _(i):
      cps0 = start(i, ra0, rb0, sa0, sb0)
      cps1 = start(i + 1, ra1, rb1, sa1, sb1)
      for cp in cps0:
        cp.wait()
      process(i, ra0, rb0)
      for cp in cps1:
        cp.wait()
      process(i + 1, ra1, rb1)

  return k


def _sc_gather_pair(K):
  @functools.partial(
      pl.kernel,
      mesh=_mesh(),
      compiler_params=pltpu.CompilerParams(use_tc_tiling_on_sc=False),
      out_type=[jax.ShapeDtypeStruct((E, K), jnp.float32),
                jax.ShapeDtypeStruct((E, K), jnp.float32)],
      scratch_types=[
          pltpu.VMEM((NCH, C), jnp.int32),
          pltpu.VMEM((NCH, C), jnp.int32),
          pltpu.VMEM((C, K), jnp.float32),
          pltpu.VMEM((C, K), jnp.float32),
          pltpu.VMEM((C, K), jnp.float32),
          pltpu.VMEM((C, K), jnp.float32),
          pltpu.SemaphoreType.DMA,
          pltpu.SemaphoreType.DMA,
          pltpu.SemaphoreType.DMA,
          pltpu.SemaphoreType.DMA,
      ],
  )
  def k(tab_hbm, dsti_hbm, srci_hbm, outd_hbm, outs_hbm,
        idxd, idxs, rd0, rs0, rd1, rs1, sa0, sb0, sa1, sb1):
    w = lax.axis_index("s") * NC + lax.axis_index("c")
    base_w = w * EW
    pltpu.sync_copy(dsti_hbm.at[w], idxd)
    pltpu.sync_copy(srci_hbm.at[w], idxs)

    def start(j, rd, rs, sa, sb):
      ca = pltpu.async_copy(tab_hbm.at[idxd.at[j]], rd, sa)
      cb = pltpu.async_copy(tab_hbm.at[idxs.at[j]], rs, sb)
      return ca, cb

    def process(j, rd, rs):
      rows = pl.ds(base_w + j * C, C)
      pltpu.sync_copy(rd, outd_hbm.at[rows])
      pltpu.sync_copy(rs, outs_hbm.at[rows])

    @pl.loop(0, NCH, step=2)
    def _(j):
      c0a, c0b = start(j, rd0, rs0, sa0, sb0)
      c1a, c1b = start(j + 1, rd1, rs1, sa1, sb1)
      c0a.wait()
      c0b.wait()
      process(j, rd0, rs0)
      c1a.wait()
      c1b.wait()
      process(j + 1, rd1, rs1)

  return k


def _zero_fill(zbuf, rows, width):
  @pl.loop(0, rows)
  def _(i):
    for l in range(width // 16):
      zbuf[i, pl.ds(l * 16, 16)] = jnp.zeros((16,), jnp.float32)


def _sc_scatter(KV, with_cnt):
  B = 2 if KV == 64 else 1
  NB = NCH // B

  @functools.partial(
      pl.kernel,
      mesh=_mesh(),
      compiler_params=pltpu.CompilerParams(use_tc_tiling_on_sc=False),
      out_type=([jax.ShapeDtypeStruct((NC, N, KV), jnp.float32),
                 jax.ShapeDtypeStruct((NC, N, 16), jnp.float32)]
                if with_cnt else
                [jax.ShapeDtypeStruct((NC, N, KV), jnp.float32)]),
      scratch_types=(
          [pltpu.VMEM((NCH, C), jnp.int32),
           pltpu.VMEM((B * C, KV), jnp.float32),
           pltpu.VMEM((B * C, KV), jnp.float32),
           pltpu.VMEM_SHARED((N, KV), jnp.float32),
           pltpu.SemaphoreType.DMA,
           pltpu.SemaphoreType.DMA,
           pltpu.SemaphoreType.DMA]
          + ([pltpu.VMEM((C, 16), jnp.float32),
              pltpu.VMEM_SHARED((N, 16), jnp.float32),
              pltpu.SemaphoreType.DMA] if with_cnt else [])),
  )
  def k(v_hbm, dsti_hbm, *rest):
    if with_cnt:
      (v_out, cnt_out, idxd, v0, v1, sh_v, sem_l0, sem_l1, sem_s,
       vones, sh_cnt, sem_c) = rest
    else:
      v_out, idxd, v0, v1, sh_v, sem_l0, sem_l1, sem_s = rest
    c = lax.axis_index("c")
    s = lax.axis_index("s")
    w = s * NC + c
    base_w = w * EW

    pltpu.sync_copy(dsti_hbm.at[w], idxd)
    _zero_fill(v0, RZ, KV)
    if with_cnt:
      _zero_fill(vones, RZ, 16)
    for m in range(RT // RZ):
      row = s * RT + m * RZ
      pltpu.sync_copy(v0.at[pl.ds(0, RZ)], sh_v.at[pl.ds(row, RZ)])
      if with_cnt:
        pltpu.sync_copy(vones, sh_cnt.at[pl.ds(row, RZ)])
    if with_cnt:
      @pl.loop(0, C)
      def _(i):
        vones[i, pl.ds(0, 16)] = jnp.ones((16,), jnp.float32)
    plsc.subcore_barrier()

    def load(b, vbuf, sem):
      return pltpu.async_copy(
          v_hbm.at[pl.ds(base_w + b * B * C, B * C)], vbuf, sem)

    def scat(b, vbuf):
      cps = []
      for t in range(B):
        j = b * B + t
        cps.append(pltpu.async_copy(vbuf.at[pl.ds(t * C, C)],
                                    sh_v.at[idxd.at[j]], sem_s, add=True))
        if with_cnt:
          cps.append(pltpu.async_copy(vones, sh_cnt.at[idxd.at[j]], sem_c,
                                      add=True))
      for cp in cps:
        cp.wait()

    @pl.loop(0, NB, step=2)
    def _(b):
      c0 = load(b, v0, sem_l0)
      c1 = load(b + 1, v1, sem_l1)
      c0.wait()
      scat(b, v0)
      c1.wait()
      scat(b + 1, v1)

    plsc.subcore_barrier()
    row = s * RT
    pltpu.sync_copy(sh_v.at[pl.ds(row, RT)], v_out.at[c, pl.ds(row, RT)])
    if with_cnt:
      pltpu.sync_copy(sh_cnt.at[pl.ds(row, RT)], cnt_out.at[c, pl.ds(row, RT)])

  return k


def _row(v):
  return v.reshape(1, -1)


def kernel(x_in, edge_index, edge_attr, params):
  p = params
  x = x_in[0]
  ea = edge_attr[0]
  ei = edge_index[0].astype(jnp.int32)
  src = ei[0].reshape(NW, NCH, C)
  dst = ei[1].reshape(NW, NCH, C)

  w1t = p['c1_W1'].T
  w1d, w1s, w1e = w1t[:D], w1t[D:2 * D], w1t[2 * D:]
  w2t = p['c2_W1'].T
  w2d, w2s, w2e = w2t[:H], w2t[H:2 * H], w2t[2 * H:]

  grid_n = N // _BN
  pa, pb, skip, gate = pl.pallas_call(
      _node_pre_body,
      grid=(grid_n,),
      in_specs=[
          pl.BlockSpec((_BN, D), lambda i: (i, 0)),
          _full((1, D)), _full((1, D)), _full((1, D)),
          _full((D, H)), _full((1, H)), _full((D, H)),
          _full((D, OD)), _full((1, OD)), _full((OD, OD)), _full((1, OD)),
      ],
      out_specs=[
          pl.BlockSpec((_BN, H), lambda i: (i, 0)),
          pl.BlockSpec((_BN, H), lambda i: (i, 0)),
          pl.BlockSpec((_BN, OD), lambda i: (i, 0)),
          pl.BlockSpec((_BN, OD), lambda i: (i, 0)),
      ],
      out_shape=[
          jax.ShapeDtypeStruct((N, H), jnp.float32),
          jax.ShapeDtypeStruct((N, H), jnp.float32),
          jax.ShapeDtypeStruct((N, OD), jnp.float32),
          jax.ShapeDtypeStruct((N, OD), jnp.float32),
      ],
  )(x, _row(p['dummy']), _row(p['ln0_g']), _row(p['ln0_b']), w1d,
    _row(p['c1_b1']), w1s, p['skip_W'].T, _row(p['skip_b']), p['gate_W'].T,
    _row(p['gate_b']))

  pn1 = _sc_gather(H)(pa, pb, dst, src)

  grid_e = E // _BE
  eenc, h1 = pl.pallas_call(
      _edge1_body,
      grid=(grid_e,),
      in_specs=[
          pl.BlockSpec((_BE, ED), lambda i: (i, 0)),
          pl.BlockSpec((_BE, H), lambda i: (i, 0)),
          _full((1, ED)), _full((1, ED)),
          _full((ED, OD)), _full((1, OD)),
          _full((OD, 2 * OD)), _full((1, 2 * OD)),
          _full((2 * OD, OD)), _full((1, OD)),
          _full((ED, ED)), _full((1, ED)), _full((1, ED)), _full((1, 1)),
          _full((OD, H)), _full((H, H)), _full((1, H)), _full((H, H)),
          _full((1, H)),
      ],
      out_specs=[
          pl.BlockSpec((_BE, OD), lambda i: (i, 0)),
          pl.BlockSpec((_BE, H), lambda i: (i, 0)),
      ],
      out_shape=[
          jax.ShapeDtypeStruct((E, OD), jnp.float32),
          jax.ShapeDtypeStruct((E, H), jnp.float32),
      ],
  )(ea, pn1, _row(p['ee_ln_g']), _row(p['ee_ln_b']), p['ee_W1'].T,
    _row(p['ee_b1']), p['ee_W2'].T, _row(p['ee_b2']), p['ee_W3'].T,
    _row(p['ee_b3']), p['ec_W1'].T, _row(p['ec_b1']), _row(p['ec_W2']),
    _row(p['ec_b2']), w1e, p['c1_W2'].T, _row(p['c1_b2']), p['c1_W3'].T,
    _row(p['c1_b3']))

  s1p, cntp = _sc_scatter(H, True)(h1, dst)
  efp, = _sc_scatter(OD, False)(eenc, dst)

  pa2, pb2 = pl.pallas_call(
      _node_mid_body,
      grid=(grid_n,),
      in_specs=[
          pl.BlockSpec((NC, _BN, H), lambda i: (0, i, 0)),
          pl.BlockSpec((NC, _BN, 16), lambda i: (0, i, 0)),
          _full((1, H)), _full((1, H)),
          _full((H, OD)), _full((1, OD)), _full((H, OD)),
      ],
      out_specs=[
          pl.BlockSpec((_BN, OD), lambda i: (i, 0)),
          pl.BlockSpec((_BN, OD), lambda i: (i, 0)),
      ],
      out_shape=[
          jax.ShapeDtypeStruct((N, OD), jnp.float32),
          jax.ShapeDtypeStruct((N, OD), jnp.float32),
      ],
  )(s1p, cntp, _row(p['ln1_g']), _row(p['ln1_b']), w2d, _row(p['c2_b1']),
    w2s)

  pn2 = _sc_gather(OD)(pa2, pb2, dst, src)

  h2 = pl.pallas_call(
      _edge2_body,
      grid=(grid_e,),
      in_specs=[
          pl.BlockSpec((_BE, OD), lambda i: (i, 0)),
          pl.BlockSpec((_BE, OD), lambda i: (i, 0)),
          _full((OD, OD)), _full((OD, OD)), _full((1, OD)),
          _full((OD, OD)), _full((1, OD)),
      ],
      out_specs=pl.BlockSpec((_BE, OD), lambda i: (i, 0)),
      out_shape=jax.ShapeDtypeStruct((E, OD), jnp.float32),
  )(pn2, eenc, w2e, p['c2_W2'].T, _row(p['c2_b2']), p['c2_W3'].T,
    _row(p['c2_b3']))

  s2p, = _sc_scatter(OD, False)(h2, dst)

  xfc, node_probs = pl.pallas_call(
      _node_final_body,
      grid=(grid_n,),
      in_specs=[
          pl.BlockSpec((NC, _BN, OD), lambda i: (0, i, 0)),
          pl.BlockSpec((NC, _BN, 16), lambda i: (0, i, 0)),
          pl.BlockSpec((NC, _BN, OD), lambda i: (0, i, 0)),
          pl.BlockSpec((_BN, OD), lambda i: (i, 0)),
          pl.BlockSpec((_BN, OD), lambda i: (i, 0)),
          _full((1, OD)), _full((1, OD)),
          _full((2 * OD, OD)), _full((1, OD)),
          _full((OD, H)), _full((1, H)),
          _full((1, H)), _full((1, 1)),
      ],
      out_specs=[
          pl.BlockSpec((_BN, 2 * OD), lambda i: (i, 0)),
          pl.BlockSpec((_BN, 1), lambda i: (i, 0)),
      ],
      out_shape=[
          jax.ShapeDtypeStruct((N, 2 * OD), jnp.float32),
          jax.ShapeDtypeStruct((N, 1), jnp.float32),
      ],
  )(s2p, cntp, efp, skip, gate, _row(p['ln2_g']), _row(p['ln2_b']),
    p['np_W1'].T, _row(p['np_b1']), p['np_W2'].T, _row(p['np_b2']),
    _row(p['np_W3']), _row(p['np_b3']))

  return (xfc, node_probs)
